# baseline (speedup 1.0000x reference)
import math

import jax
import jax.numpy as jnp
from jax import lax
from jax.experimental import pallas as pl
from jax.experimental.pallas import tpu as pltpu
from jax.experimental.pallas import tpu_sc as plsc

_INFO = plsc.get_sparse_core_info()
_NC = _INFO.num_cores
_NS = _INFO.num_subcores
_NW = _NC * _NS


def _make_sc_kernel(B, S, V, D):
    assert S % _NW == 0
    seq_per_w = S // _NW
    chunk = 16
    assert seq_per_w % chunk == 0
    nchunk = seq_per_w // chunk
    scale = float(math.sqrt(D))
    jblocks = D // 16
    mesh = plsc.VectorSubcoreMesh(core_axis_name="c", subcore_axis_name="s")

    def body(tok_hbm, table_hbm, pe_hbm, out_hbm, idx_v, pe_v, rows_v,
             gsem, psem):
        wid = lax.axis_index("s") * _NC + lax.axis_index("c")

        def chunk_body(p, _):
            sbase = pl.multiple_of(wid * seq_per_w + p * chunk, chunk)
            pe_cp = pltpu.async_copy(pe_hbm.at[pl.ds(sbase, chunk)], pe_v,
                                     psem)
            for b in range(B):
                pltpu.sync_copy(tok_hbm.at[pl.ds(b * S + sbase, chunk)],
                                idx_v.at[b])
            cps = [pltpu.async_copy(table_hbm.at[idx_v.at[b]], rows_v.at[b],
                                    gsem) for b in range(B)]
            for cp in cps:
                cp.wait()
            pe_cp.wait()

            def r_body(r, _):
                def j_body(j, _):
                    o = pl.multiple_of(j * 16, 16)
                    pec = pe_v[r, pl.ds(o, 16)]
                    for b in range(B):
                        rows_v[b, r, pl.ds(o, 16)] = (
                            rows_v[b, r, pl.ds(o, 16)] * scale + pec)
                    return 0

                return lax.fori_loop(0, jblocks, j_body, 0, unroll=4)

            lax.fori_loop(0, chunk, r_body, 0)
            for b in range(B):
                pltpu.sync_copy(rows_v.at[b],
                                out_hbm.at[pl.ds(b * S + sbase, chunk)])
            return 0

        lax.fori_loop(0, nchunk, chunk_body, 0)

    return pl.kernel(
        body,
        out_type=jax.ShapeDtypeStruct((B * S, D), jnp.float32),
        mesh=mesh,
        scratch_types=[
            pltpu.VMEM((B, chunk), jnp.int32),
            pltpu.VMEM((chunk, D), jnp.float32),
            pltpu.VMEM((B, chunk, D), jnp.float32),
            pltpu.SemaphoreType.DMA,
            pltpu.SemaphoreType.DMA,
        ],
    )


def kernel(token_tensor, emb_table, pe):
    B, S = token_tensor.shape
    V, D = emb_table.shape
    tok = token_tensor.reshape(B * S).astype(jnp.int32)
    out = _make_sc_kernel(B, S, V, D)(tok, emb_table, pe)
    return out.reshape(B, S, D)


# speedup vs baseline: 2.7660x; 2.7660x over previous
import math

import jax
import jax.numpy as jnp
from jax import lax
from jax.experimental import pallas as pl
from jax.experimental.pallas import tpu as pltpu
from jax.experimental.pallas import tpu_sc as plsc

_INFO = plsc.get_sparse_core_info()
_NC = _INFO.num_cores
_NS = _INFO.num_subcores
_NW = _NC * _NS


def _make_sc_kernel(B, S, V, D):
    assert S % _NW == 0
    seq_per_w = S // _NW
    chunk = 8
    assert seq_per_w % chunk == 0
    nchunk = seq_per_w // chunk
    rows_per_chunk = B * chunk
    scale = float(math.sqrt(D))
    jblocks = D // 16
    jshift = jblocks.bit_length() - 1
    scratch = [
        pltpu.VMEM((nchunk, rows_per_chunk), jnp.int32),
        pltpu.VMEM((rows_per_chunk, D), jnp.float32),
        pltpu.VMEM((rows_per_chunk, D), jnp.float32),
        pltpu.VMEM((chunk, D), jnp.float32),
        pltpu.VMEM((chunk, D), jnp.float32),
        pltpu.SemaphoreType.DMA,
        pltpu.SemaphoreType.DMA,
        pltpu.SemaphoreType.DMA,
        pltpu.SemaphoreType.DMA,
        pltpu.SemaphoreType.DMA,
        pltpu.SemaphoreType.DMA,
    ]
    mesh = plsc.VectorSubcoreMesh(core_axis_name="c", subcore_axis_name="s")

    def body(tok_perm_hbm, table_hbm, pe_hbm, out_hbm, idx_v,
             rows0, rows1, pe0, pe1, gsem0, gsem1, psem0, psem1,
             osem0, osem1):
        wid = lax.axis_index("s") * _NC + lax.axis_index("c")
        sbase0 = pl.multiple_of(wid * seq_per_w, seq_per_w)
        rows = (rows0, rows1)
        peb = (pe0, pe1)
        gsem = (gsem0, gsem1)
        psem = (psem0, psem1)
        osem = (osem0, osem1)

        pltpu.sync_copy(tok_perm_hbm.at[wid], idx_v)

        def start_in(p):
            s = p % 2
            g = pltpu.async_copy(table_hbm.at[idx_v.at[p]], rows[s], gsem[s])
            q = pltpu.async_copy(
                pe_hbm.at[pl.ds(pl.multiple_of(sbase0 + p * chunk, chunk),
                                chunk)],
                peb[s], psem[s])
            return g, q

        def start_out(p):
            s = p % 2
            return [
                pltpu.async_copy(
                    rows[s].at[pl.ds(b * chunk, chunk)],
                    out_hbm.at[pl.ds(
                        pl.multiple_of(b * S + sbase0 + p * chunk, chunk),
                        chunk)],
                    osem[s])
                for b in range(B)
            ]

        pending_in = {0: start_in(0)}
        pending_out = {}
        for p in range(nchunk):
            s = p % 2
            if p + 1 < nchunk:
                if p - 1 in pending_out:
                    for cp in pending_out.pop(p - 1):
                        cp.wait()
                pending_in[p + 1] = start_in(p + 1)
            g, q = pending_in.pop(p)
            g.wait()
            q.wait()

            @plsc.parallel_loop(0, chunk * jblocks, unroll=4)
            def _(i):
                r = i >> jshift
                o = pl.multiple_of((i & (jblocks - 1)) * 16, 16)
                pec = peb[s][r, pl.ds(o, 16)]
                for b in range(B):
                    rows[s][b * chunk + r, pl.ds(o, 16)] = (
                        rows[s][b * chunk + r, pl.ds(o, 16)] * scale + pec)

            pending_out[p] = start_out(p)
        for p in sorted(pending_out):
            for cp in pending_out.pop(p):
                cp.wait()

    return pl.kernel(
        body,
        out_type=jax.ShapeDtypeStruct((B * S, D), jnp.float32),
        mesh=mesh,
        scratch_types=scratch,
    )


def kernel(token_tensor, emb_table, pe):
    B, S = token_tensor.shape
    V, D = emb_table.shape
    seq_per_w = S // _NW
    chunk = 8
    nchunk = seq_per_w // chunk
    tok_perm = (token_tensor.astype(jnp.int32)
                .reshape(B, _NW, nchunk, chunk)
                .transpose(1, 2, 0, 3)
                .reshape(_NW, nchunk, B * chunk))
    out = _make_sc_kernel(B, S, V, D)(tok_perm, emb_table, pe)
    return out.reshape(B, S, D)


# speedup vs baseline: 2.7809x; 1.0054x over previous
import math

import jax
import jax.numpy as jnp
from jax import lax
from jax.experimental import pallas as pl
from jax.experimental.pallas import tpu as pltpu
from jax.experimental.pallas import tpu_sc as plsc

_INFO = plsc.get_sparse_core_info()
_NC = _INFO.num_cores
_NS = _INFO.num_subcores
_NW = _NC * _NS


def _make_sc_kernel(B, S, V, D):
    assert S % _NW == 0
    seq_per_w = S // _NW
    chunk = 8
    assert seq_per_w % chunk == 0
    nchunk = seq_per_w // chunk
    rows_per_chunk = B * chunk
    scale = float(math.sqrt(D))
    jblocks = D // 16
    jshift = jblocks.bit_length() - 1
    nbuf = 3
    scratch = (
        [pltpu.VMEM((nchunk, rows_per_chunk), jnp.int32)]
        + [pltpu.VMEM((rows_per_chunk, D), jnp.float32)] * nbuf
        + [pltpu.VMEM((chunk, D), jnp.float32)] * nbuf
        + [pltpu.SemaphoreType.DMA] * (3 * nbuf)
    )
    mesh = plsc.VectorSubcoreMesh(core_axis_name="c", subcore_axis_name="s")

    def body(tok_perm_hbm, table_hbm, pe_hbm, out_hbm, idx_v, *bufs):
        rows = bufs[:nbuf]
        peb = bufs[nbuf:2 * nbuf]
        gsem = bufs[2 * nbuf:3 * nbuf]
        psem = bufs[3 * nbuf:4 * nbuf]
        osem = bufs[4 * nbuf:5 * nbuf]
        wid = lax.axis_index("s") * _NC + lax.axis_index("c")
        sbase0 = pl.multiple_of(wid * seq_per_w, seq_per_w)

        pltpu.sync_copy(tok_perm_hbm.at[wid], idx_v)

        def start_in(p):
            s = p % nbuf
            g = pltpu.async_copy(table_hbm.at[idx_v.at[p]], rows[s], gsem[s])
            q = pltpu.async_copy(
                pe_hbm.at[pl.ds(pl.multiple_of(sbase0 + p * chunk, chunk),
                                chunk)],
                peb[s], psem[s])
            return g, q

        def start_out(p):
            s = p % nbuf
            return [
                pltpu.async_copy(
                    rows[s].at[pl.ds(b * chunk, chunk)],
                    out_hbm.at[pl.ds(
                        pl.multiple_of(b * S + sbase0 + p * chunk, chunk),
                        chunk)],
                    osem[s])
                for b in range(B)
            ]

        depth = nbuf - 1
        pending_in = {p: start_in(p) for p in range(min(depth, nchunk))}
        pending_out = {}
        for p in range(nchunk):
            s = p % nbuf
            g, q = pending_in.pop(p)
            g.wait()
            q.wait()

            @plsc.parallel_loop(0, chunk * jblocks, unroll=4)
            def _(i):
                r = i >> jshift
                o = pl.multiple_of((i & (jblocks - 1)) * 16, 16)
                pec = peb[s][r, pl.ds(o, 16)]
                for b in range(B):
                    rows[s][b * chunk + r, pl.ds(o, 16)] = (
                        rows[s][b * chunk + r, pl.ds(o, 16)] * scale + pec)

            pending_out[p] = start_out(p)
            nxt = p + depth
            if nxt < nchunk:
                prev = nxt - nbuf
                if prev in pending_out:
                    for cp in pending_out.pop(prev):
                        cp.wait()
                pending_in[nxt] = start_in(nxt)
        for p in sorted(pending_out):
            for cp in pending_out.pop(p):
                cp.wait()

    return pl.kernel(
        body,
        out_type=jax.ShapeDtypeStruct((B * S, D), jnp.float32),
        mesh=mesh,
        scratch_types=scratch,
    )


def kernel(token_tensor, emb_table, pe):
    B, S = token_tensor.shape
    V, D = emb_table.shape
    seq_per_w = S // _NW
    chunk = 8
    nchunk = seq_per_w // chunk
    tok_perm = (token_tensor.astype(jnp.int32)
                .reshape(B, _NW, nchunk, chunk)
                .transpose(1, 2, 0, 3)
                .reshape(_NW, nchunk, B * chunk))
    out = _make_sc_kernel(B, S, V, D)(tok_perm, emb_table, pe)
    return out.reshape(B, S, D)


# speedup vs baseline: 2.7829x; 1.0007x over previous
import math

import jax
import jax.numpy as jnp
from jax import lax
from jax.experimental import pallas as pl
from jax.experimental.pallas import tpu as pltpu
from jax.experimental.pallas import tpu_sc as plsc

_INFO = plsc.get_sparse_core_info()
_NC = _INFO.num_cores
_NS = _INFO.num_subcores
_NW = _NC * _NS


def _make_sc_kernel(B, S, V, D):
    assert S % _NW == 0
    seq_per_w = S // _NW
    chunk = 8
    assert seq_per_w % chunk == 0
    nchunk = seq_per_w // chunk
    rows_per_chunk = B * chunk
    scale = float(math.sqrt(D))
    jblocks = D // 16
    jshift = jblocks.bit_length() - 1
    nbuf = 3
    scratch = (
        [pltpu.VMEM((nchunk, rows_per_chunk), jnp.int32)]
        + [pltpu.VMEM((rows_per_chunk, D), jnp.float32)] * nbuf
        + [pltpu.VMEM((chunk, D), jnp.float32)] * nbuf
        + [pltpu.SemaphoreType.DMA] * (3 * nbuf)
    )
    mesh = plsc.VectorSubcoreMesh(core_axis_name="c", subcore_axis_name="s")

    def body(tok_perm_hbm, table_hbm, pe_hbm, out_hbm, idx_v, *bufs):
        rows = bufs[:nbuf]
        peb = bufs[nbuf:2 * nbuf]
        gsem = bufs[2 * nbuf:3 * nbuf]
        psem = bufs[3 * nbuf:4 * nbuf]
        osem = bufs[4 * nbuf:5 * nbuf]
        wid = lax.axis_index("s") * _NC + lax.axis_index("c")
        sbase0 = pl.multiple_of(wid * seq_per_w, seq_per_w)

        pltpu.sync_copy(tok_perm_hbm.at[wid], idx_v)

        nsplit = 4
        gsub = rows_per_chunk // nsplit

        def start_in(p):
            s = p % nbuf
            gs = [
                pltpu.async_copy(
                    table_hbm.at[idx_v.at[p, pl.ds(k * gsub, gsub)]],
                    rows[s].at[pl.ds(k * gsub, gsub)],
                    gsem[s])
                for k in range(nsplit)
            ]
            q = pltpu.async_copy(
                pe_hbm.at[pl.ds(pl.multiple_of(sbase0 + p * chunk, chunk),
                                chunk)],
                peb[s], psem[s])
            return gs, q

        def start_out(p):
            s = p % nbuf
            return [
                pltpu.async_copy(
                    rows[s].at[pl.ds(b * chunk, chunk)],
                    out_hbm.at[pl.ds(
                        pl.multiple_of(b * S + sbase0 + p * chunk, chunk),
                        chunk)],
                    osem[s])
                for b in range(B)
            ]

        depth = nbuf - 1
        pending_in = {p: start_in(p) for p in range(min(depth, nchunk))}
        pending_out = {}
        for p in range(nchunk):
            s = p % nbuf
            gs, q = pending_in.pop(p)
            for g in gs:
                g.wait()
            q.wait()

            if True:
                @plsc.parallel_loop(0, chunk * jblocks, unroll=4)
                def _(i):
                    r = i >> jshift
                    o = pl.multiple_of((i & (jblocks - 1)) * 16, 16)
                    pec = peb[s][r, pl.ds(o, 16)]
                    for b in range(B):
                        rows[s][b * chunk + r, pl.ds(o, 16)] = (
                            rows[s][b * chunk + r, pl.ds(o, 16)] * scale + pec)

            pending_out[p] = start_out(p)
            nxt = p + depth
            if nxt < nchunk:
                prev = nxt - nbuf
                if prev in pending_out:
                    for cp in pending_out.pop(prev):
                        cp.wait()
                pending_in[nxt] = start_in(nxt)
        for p in sorted(pending_out):
            for cp in pending_out.pop(p):
                cp.wait()

    return pl.kernel(
        body,
        out_type=jax.ShapeDtypeStruct((B * S, D), jnp.float32),
        mesh=mesh,
        scratch_types=scratch,
    )


def kernel(token_tensor, emb_table, pe):
    B, S = token_tensor.shape
    V, D = emb_table.shape
    seq_per_w = S // _NW
    chunk = 8
    nchunk = seq_per_w // chunk
    tok_perm = (token_tensor.astype(jnp.int32)
                .reshape(B, _NW, nchunk, chunk)
                .transpose(1, 2, 0, 3)
                .reshape(_NW, nchunk, B * chunk))
    out = _make_sc_kernel(B, S, V, D)(tok_perm, emb_table, pe)
    return out.reshape(B, S, D)
